# baseline (speedup 1.0000x reference)
import functools

import jax
import jax.numpy as jnp
from jax import lax
from jax.experimental import pallas as pl
from jax.experimental.pallas import tpu as pltpu
from jax.experimental.pallas import tpu_sc as plsc

OUT_DIM = 64
SEQ = 200
PAD_IDX = 0
NUM_CORES = 2
NUM_SUBCORES = 16
NUM_WORKERS = NUM_CORES * NUM_SUBCORES
CHUNK_B = 4
CHUNK = CHUNK_B * SEQ


def _emb_body(idx_hbm, table_hbm, out_hbm, idx_v, rows_v, sem, *, b_per_w):
  wid = lax.axis_index("s") * NUM_CORES + lax.axis_index("c")
  base = wid * b_per_w
  nchunk = b_per_w // CHUNK

  def chunk_body(k, carry):
    off = base + k * CHUNK
    pltpu.sync_copy(idx_hbm.at[pl.ds(off, CHUNK)], idx_v)
    pltpu.async_copy(table_hbm.at[idx_v], rows_v, sem).wait()

    def grp(g, tot):
      v = idx_v[pl.ds(g * 16, 16)]
      return tot + jnp.sum((v == PAD_IDX).astype(jnp.int32))

    tot = lax.fori_loop(0, CHUNK // 16, grp, jnp.int32(0))

    @pl.when(tot > 0)
    def _():
      zeros = jnp.zeros((16,), jnp.float32)

      def fix_grp(g, c):
        v = idx_v[pl.ds(g * 16, 16)]
        cnt = jnp.sum((v == PAD_IDX).astype(jnp.int32))

        @pl.when(cnt > 0)
        def _():
          for j in range(16):
            @pl.when(v[j] == PAD_IDX)
            def _():
              for q in range(OUT_DIM // 16):
                rows_v[g * 16 + j, pl.ds(q * 16, 16)] = zeros

        return c

      lax.fori_loop(0, CHUNK // 16, fix_grp, 0)

    bent = off // SEQ
    for e in range(CHUNK_B):
      pltpu.sync_copy(rows_v.at[pl.ds(e * SEQ, SEQ)], out_hbm.at[bent + e])
    return carry

  lax.fori_loop(0, nchunk, chunk_body, 0)


def kernel(inputs, embeddings):
  b, l = inputs.shape
  n = b * l
  assert n % NUM_WORKERS == 0
  b_per_w = n // NUM_WORKERS
  assert b_per_w % CHUNK == 0

  idx = inputs.reshape(n).astype(jnp.int32)

  mesh = plsc.VectorSubcoreMesh(
      core_axis_name="c", subcore_axis_name="s", num_cores=NUM_CORES,
      num_subcores=NUM_SUBCORES)
  fn = pl.kernel(
      functools.partial(_emb_body, b_per_w=b_per_w),
      out_type=jax.ShapeDtypeStruct((b, l, OUT_DIM), jnp.float32),
      mesh=mesh,
      scratch_types=[
          pltpu.VMEM((CHUNK,), jnp.int32),
          pltpu.VMEM((CHUNK, OUT_DIM), jnp.float32),
          pltpu.SemaphoreType.DMA,
      ],
      compiler_params=pltpu.CompilerParams(
          use_tc_tiling_on_sc=False, needs_layout_passes=False),
  )
  return fn(idx, embeddings)


# speedup vs baseline: 1.2315x; 1.2315x over previous
import functools

import jax
import jax.numpy as jnp
from jax import lax
from jax.experimental import pallas as pl
from jax.experimental.pallas import tpu as pltpu
from jax.experimental.pallas import tpu_sc as plsc

OUT_DIM = 64
PAD_W = 128
SEQ = 200
PAD_IDX = 0
NUM_CORES = 2
NUM_SUBCORES = 16
NUM_WORKERS = NUM_CORES * NUM_SUBCORES
CHUNK_B = 4
CHUNK = CHUNK_B * SEQ


def _emb_body(idx_hbm, table_hbm, out_hbm, idx_v, rows_v, sem, *, b_per_w):
  wid = lax.axis_index("s") * NUM_CORES + lax.axis_index("c")
  base = wid * b_per_w
  nchunk = b_per_w // CHUNK

  def chunk_body(k, carry):
    off = base + k * CHUNK
    pltpu.sync_copy(idx_hbm.at[pl.ds(off, CHUNK)], idx_v)
    pltpu.async_copy(table_hbm.at[idx_v], rows_v, sem).wait()

    def grp(g, tot):
      v = idx_v[pl.ds(g * 16, 16)]
      return tot + jnp.sum((v == PAD_IDX).astype(jnp.int32))

    tot = lax.fori_loop(0, CHUNK // 16, grp, jnp.int32(0))

    @pl.when(tot > 0)
    def _():
      zeros = jnp.zeros((16,), jnp.float32)

      def fix_grp(g, c):
        v = idx_v[pl.ds(g * 16, 16)]
        cnt = jnp.sum((v == PAD_IDX).astype(jnp.int32))

        @pl.when(cnt > 0)
        def _():
          for j in range(16):
            @pl.when(v[j] == PAD_IDX)
            def _():
              for q in range(OUT_DIM // 16):
                rows_v[g * 16 + j, pl.ds(q * 16, 16)] = zeros

        return c

      lax.fori_loop(0, CHUNK // 16, fix_grp, 0)

    bent = off // SEQ
    for e in range(CHUNK_B):
      pltpu.sync_copy(rows_v.at[pl.ds(e * SEQ, SEQ)], out_hbm.at[bent + e])
    return carry

  lax.fori_loop(0, nchunk, chunk_body, 0)


def kernel(inputs, embeddings):
  b, l = inputs.shape
  n = b * l
  assert n % NUM_WORKERS == 0
  b_per_w = n // NUM_WORKERS
  assert b_per_w % CHUNK == 0

  idx = inputs.reshape(n).astype(jnp.int32)
  emb_pad = jnp.pad(embeddings, ((0, 0), (0, PAD_W - OUT_DIM)))

  mesh = plsc.VectorSubcoreMesh(
      core_axis_name="c", subcore_axis_name="s", num_cores=NUM_CORES,
      num_subcores=NUM_SUBCORES)
  fn = pl.kernel(
      functools.partial(_emb_body, b_per_w=b_per_w),
      out_type=jax.ShapeDtypeStruct((b, l, PAD_W), jnp.float32),
      mesh=mesh,
      scratch_types=[
          pltpu.VMEM((CHUNK,), jnp.int32),
          pltpu.VMEM((CHUNK, PAD_W), jnp.float32),
          pltpu.SemaphoreType.DMA,
      ],
      compiler_params=pltpu.CompilerParams(
          use_tc_tiling_on_sc=False, needs_layout_passes=False),
  )
  return fn(idx, emb_pad)[:, :, :OUT_DIM]


# speedup vs baseline: 1.3201x; 1.0720x over previous
import functools

import jax
import jax.numpy as jnp
from jax import lax
from jax.experimental import pallas as pl
from jax.experimental.pallas import tpu as pltpu
from jax.experimental.pallas import tpu_sc as plsc

OUT_DIM = 64
PAD_W = 128
SEQ = 200
PAD_IDX = 0
NUM_CORES = 2
NUM_SUBCORES = 16
NUM_WORKERS = NUM_CORES * NUM_SUBCORES
CHUNK_B = 4
CHUNK = CHUNK_B * SEQ


def _emb_body(idx_hbm, table_hbm, out_hbm, idx_v, rows_v, sem, *, b_per_w):
  wid = lax.axis_index("s") * NUM_CORES + lax.axis_index("c")
  base = wid * b_per_w
  nchunk = b_per_w // CHUNK

  def chunk_body(k, carry):
    off = base + k * CHUNK
    pltpu.sync_copy(idx_hbm.at[pl.ds(off, CHUNK)], idx_v)
    pltpu.async_copy(table_hbm.at[idx_v], rows_v, sem).wait()

    def grp(g, tot):
      v = idx_v[pl.ds(g * 16, 16)]
      return tot + jnp.sum((v == PAD_IDX).astype(jnp.int32))

    tot = lax.fori_loop(0, CHUNK // 16, grp, jnp.int32(0))

    @pl.when(tot > 0)
    def _():
      zeros = jnp.zeros((16,), jnp.float32)

      def fix_grp(g, c):
        v = idx_v[pl.ds(g * 16, 16)]
        cnt = jnp.sum((v == PAD_IDX).astype(jnp.int32))

        @pl.when(cnt > 0)
        def _():
          for j in range(16):
            @pl.when(v[j] == PAD_IDX)
            def _():
              for q in range(OUT_DIM // 16):
                rows_v[g * 16 + j, pl.ds(q * 16, 16)] = zeros

        return c

      lax.fori_loop(0, CHUNK // 16, fix_grp, 0)

    bent = off // SEQ
    for e in range(CHUNK_B):
      pltpu.sync_copy(rows_v.at[pl.ds(e * SEQ, SEQ)],
                      out_hbm.at[bent + e, :, pl.ds(0, OUT_DIM)])
    return carry

  lax.fori_loop(0, nchunk, chunk_body, 0)


def kernel(inputs, embeddings):
  b, l = inputs.shape
  n = b * l
  assert n % NUM_WORKERS == 0
  b_per_w = n // NUM_WORKERS
  assert b_per_w % CHUNK == 0
  v = embeddings.shape[0]

  idx = inputs.reshape(n).astype(jnp.int32)
  emb2 = embeddings.reshape(v // 2, 2 * OUT_DIM)
  emb2 = lax.optimization_barrier(emb2)
  emb_lin = emb2.reshape(v, OUT_DIM)

  mesh = plsc.VectorSubcoreMesh(
      core_axis_name="c", subcore_axis_name="s", num_cores=NUM_CORES,
      num_subcores=NUM_SUBCORES)
  fn = pl.kernel(
      functools.partial(_emb_body, b_per_w=b_per_w),
      out_type=jax.ShapeDtypeStruct((b, l, PAD_W), jnp.float32),
      mesh=mesh,
      scratch_types=[
          pltpu.VMEM((CHUNK,), jnp.int32),
          pltpu.VMEM((CHUNK, OUT_DIM), jnp.float32),
          pltpu.SemaphoreType.DMA,
      ],
      compiler_params=pltpu.CompilerParams(
          use_tc_tiling_on_sc=False, needs_layout_passes=False),
  )
  return fn(idx, emb_lin)[:, :, :OUT_DIM]
